# baseline (speedup 1.0000x reference)
import jax
import jax.numpy as jnp
from jax import lax
from jax.experimental import pallas as pl
from jax.experimental.pallas import tpu as pltpu
from jax.experimental.pallas import tpu_sc as plsc

NB, SEQ = 16384, 20
DIM = 64
NC, NS = 2, 16
NW = NC * NS
ROWS_PER_W = NB * SEQ // NW
BROW_PER_W = NB // NW
K = 8
CHUNK = K * SEQ
NCHUNK = BROW_PER_W // K
NBUF = 4
LAG = 2
NGROUP = NCHUNK // NBUF
CHUNK_BYTES = CHUNK * DIM * 4


def _emb_body(idx_hbm, table_hbm, out_hbm, idx_v, rows_v, *sems):
    gsem = sems[:NBUF]
    osem = sems[NBUF:]
    wid = lax.axis_index("s") * NC + lax.axis_index("c")
    pltpu.sync_copy(idx_hbm.at[wid], idx_v)

    def enqueue_rows(c, b):
        def vgroup(v, carry):
            vec = idx_v[pl.ds(c * CHUNK + v * 16, 16)]
            for j in range(16):
                pltpu.async_copy(table_hbm.at[vec[j]],
                                 rows_v.at[b, v * 16 + j], gsem[b])
            return carry
        lax.fori_loop(0, CHUNK // 16, vgroup, 0)

    def drain_rows(bo):
        def dwait(t, carry):
            for _ in range(16):
                pltpu.make_async_copy(table_hbm.at[0],
                                      rows_v.at[bo, 0],
                                      gsem[bo]).wait()
            return carry
        lax.fori_loop(0, CHUNK // 16, dwait, 0)

    def group(g, carry):
        for b in range(NBUF):
            c = g * NBUF + b
            co = c - LAG
            bo = (b - LAG) % NBUF

            @pl.when(c < NCHUNK)
            def _():
                enqueue_rows(c, b)

            @pl.when((co >= 0) & (co < NCHUNK))
            def _():
                drain_rows(bo)
        return carry

    lax.fori_loop(0, NGROUP + 1, group, 0)


@jax.jit
def _embed(idx, table):
    mesh = plsc.VectorSubcoreMesh(core_axis_name="c", subcore_axis_name="s")
    return pl.kernel(
        _emb_body,
        mesh=mesh,
        out_type=jax.ShapeDtypeStruct((NB, SEQ, DIM), jnp.float32),
        scratch_types=[
            pltpu.VMEM((ROWS_PER_W,), jnp.int32),
            pltpu.VMEM((NBUF, CHUNK, DIM), jnp.float32),
        ] + [pltpu.SemaphoreType.DMA] * (2 * NBUF),
    )(idx, table)


def kernel(token_ids, weight):
    idx = token_ids.reshape(NW, ROWS_PER_W).astype(jnp.int32)
    return _embed(idx, weight)


# speedup vs baseline: 1.0554x; 1.0554x over previous
import jax
import jax.numpy as jnp
from jax import lax
from jax.experimental import pallas as pl
from jax.experimental.pallas import tpu as pltpu
from jax.experimental.pallas import tpu_sc as plsc

NB, SEQ = 16384, 20
DIM = 64
NC, NS = 2, 16
NW = NC * NS
ROWS_PER_W = NB * SEQ // NW
BROW_PER_W = NB // NW
K = 8
CHUNK = K * SEQ
NCHUNK = BROW_PER_W // K
NBUF = 4
LAG = 2
NGROUP = NCHUNK // NBUF
CHUNK_BYTES = CHUNK * DIM * 4


def _emb_body(idx_hbm, table_hbm, out_hbm, idx_v, rows_v, *sems):
    gsem = sems[:NBUF]
    osem = sems[NBUF:]
    wid = lax.axis_index("s") * NC + lax.axis_index("c")
    pltpu.sync_copy(idx_hbm.at[wid], idx_v)

    rows4 = rows_v.reshape(NBUF, K, SEQ, 2 * DIM)

    def enqueue_rows(c, b):
        def vgroup(v, carry):
            vec = idx_v[pl.ds(c * CHUNK + v * 16, 16)]
            for j in range(16):
                pltpu.async_copy(table_hbm.at[vec[j]],
                                 rows_v.at[b, v * 16 + j, pl.ds(0, DIM)],
                                 gsem[b])
            return carry
        lax.fori_loop(0, CHUNK // 16, vgroup, 0)

    def drain_rows(bo):
        def dwait(t, carry):
            for _ in range(16):
                pltpu.make_async_copy(table_hbm.at[0],
                                      rows_v.at[bo, 0, pl.ds(0, DIM)],
                                      gsem[bo]).wait()
            return carry
        lax.fori_loop(0, CHUNK // 16, dwait, 0)

    def writeback(c, b):
        gb = wid * BROW_PER_W + c * K
        return pltpu.make_async_copy(
            rows4.at[b], out_hbm.at[pl.ds(gb, K)], osem[b])

    def group(g, carry):
        for b in range(NBUF):
            c = g * NBUF + b
            co = c - LAG
            bo = (b - LAG) % NBUF

            @pl.when(c < NCHUNK)
            def _():
                @pl.when(c >= NBUF)
                def _():
                    writeback(c - NBUF, b).wait()
                enqueue_rows(c, b)

            @pl.when((co >= 0) & (co < NCHUNK))
            def _():
                drain_rows(bo)
                writeback(co, bo).start()
        return carry

    lax.fori_loop(0, NGROUP + 1, group, 0)
    for b in range(NBUF):
        writeback(0, b).wait()


@jax.jit
def _embed(idx, table):
    mesh = plsc.VectorSubcoreMesh(core_axis_name="c", subcore_axis_name="s")
    return pl.kernel(
        _emb_body,
        mesh=mesh,
        out_type=jax.ShapeDtypeStruct((NB, SEQ, 2 * DIM), jnp.float32),
        scratch_types=[
            pltpu.VMEM((ROWS_PER_W,), jnp.int32),
            pltpu.VMEM((NBUF, CHUNK, 2 * DIM), jnp.float32),
        ] + [pltpu.SemaphoreType.DMA] * (2 * NBUF),
    )(idx, table)


def kernel(token_ids, weight):
    idx = token_ids.reshape(NW, ROWS_PER_W).astype(jnp.int32)
    return _embed(idx, weight)[:, :, :DIM]
